# baseline (speedup 1.0000x reference)
import functools

import jax
import jax.numpy as jnp
from jax import lax
from jax.experimental import pallas as pl
from jax.experimental.pallas import tpu as pltpu
from jax.experimental.pallas import tpu_sc as plsc

N = 10000
E = 320000
NC, NS = 2, 16
NW = NC * NS
CHUNK = 128
NCHUNKS = E // CHUNK
NP = N + 16
STRIPE = 632
STRIPE_LAST = NP - STRIPE * (NS - 1)
DEG_COL = 128
RD, ID = 4, 8

_B = 2000
_GRID = N // _B


def _pad16(d):
    return -(-d // 16) * 16


def _make_agg(dpad, split):
    mesh = plsc.VectorSubcoreMesh(core_axis_name="c", subcore_axis_name="s")
    nworkers = NS if split else NW
    cnt = NCHUNKS // nworkers
    rem = NCHUNKS % nworkers

    def body(*refs):
        if split:
            (g0_hbm, g1_hbm, ei_hbm, zeros_hbm, out_hbm,
             acc, idx, rows, isem, gsem, ssem) = refs
        else:
            (g0_hbm, ei_hbm, zeros_hbm, out_hbm,
             acc, idx, rows, isem, gsem, ssem) = refs
        c = lax.axis_index("c")
        s = lax.axis_index("s")
        w = s if split else s * NC + c
        st = w * cnt + jnp.minimum(w, rem)
        row0 = s * STRIPE

        pltpu.sync_copy(zeros_hbm.at[pl.ds(row0, STRIPE_LAST)],
                        acc.at[pl.ds(row0, STRIPE_LAST)])

        @pl.when(s < NS - 1)
        def _zero_rest():
            pltpu.sync_copy(
                zeros_hbm.at[pl.ds(row0 + STRIPE_LAST, STRIPE - STRIPE_LAST)],
                acc.at[pl.ds(row0 + STRIPE_LAST, STRIPE - STRIPE_LAST)])

        def run(g_hbm):
            def idx_load(j):
                pltpu.async_copy(ei_hbm.at[st + j], idx.at[j % ID], isem)

            def idx_wait(j):
                pltpu.make_async_copy(ei_hbm.at[st + j], idx.at[j % ID],
                                      isem).wait()

            def gather_start(j):
                pltpu.async_copy(g_hbm.at[idx.at[j % ID, 0]], rows.at[j % RD],
                                 gsem)

            def gather_wait(j):
                pltpu.make_async_copy(g_hbm.at[idx.at[j % ID, 0]],
                                      rows.at[j % RD], gsem).wait()

            def scat_start(j):
                pltpu.async_copy(rows.at[j % RD], acc.at[idx.at[j % ID, 1]],
                                 ssem, add=True)

            def scat_wait(j):
                pltpu.make_async_copy(rows.at[j % RD],
                                      acc.at[idx.at[j % ID, 1]], ssem).wait()

            for t in range(4):
                idx_load(t)
            idx_wait(0)
            gather_start(0)
            idx_wait(1)
            gather_start(1)
            plsc.subcore_barrier()

            for j in (0, 1):
                idx_wait(j + 2)
                gather_start(j + 2)
                idx_load(j + 4)
                gather_wait(j)
                scat_start(j)

            def step(j, carry):
                scat_wait(j - 2)
                idx_wait(j + 2)
                gather_start(j + 2)
                idx_load(j + 4)
                gather_wait(j)
                scat_start(j)
                return carry

            lax.fori_loop(2, cnt - 4, step, 0, unroll=4)

            for j in range(cnt - 4, cnt):
                scat_wait(j - 2)
                if j + 2 < cnt:
                    idx_wait(j + 2)
                    gather_start(j + 2)
                gather_wait(j)
                scat_start(j)
            scat_wait(cnt - 2)
            scat_wait(cnt - 1)

            @pl.when(w < rem)
            def _extra():
                idx_load(cnt)
                idx_wait(cnt)
                gather_start(cnt)
                gather_wait(cnt)
                scat_start(cnt)
                scat_wait(cnt)

        if split:
            @pl.when(c == 0)
            def _run0():
                run(g0_hbm)

            @pl.when(c == 1)
            def _run1():
                run(g1_hbm)
        else:
            run(g0_hbm)

        plsc.subcore_barrier()
        pltpu.sync_copy(acc.at[pl.ds(row0, STRIPE_LAST)],
                        out_hbm.at[c, pl.ds(row0, STRIPE_LAST)])

        @pl.when(s < NS - 1)
        def _out_rest():
            pltpu.sync_copy(
                acc.at[pl.ds(row0 + STRIPE_LAST, STRIPE - STRIPE_LAST)],
                out_hbm.at[c, pl.ds(row0 + STRIPE_LAST, STRIPE - STRIPE_LAST)])

    in_types = [jax.ShapeDtypeStruct((NP, dpad), jnp.float32)] * (2 if split else 1)
    del in_types

    return pl.kernel(
        body,
        out_type=jax.ShapeDtypeStruct((NC, NP, dpad), jnp.float32),
        mesh=mesh,
        compiler_params=pltpu.CompilerParams(use_tc_tiling_on_sc=False),
        scratch_types=[
            pltpu.VMEM_SHARED((NP, dpad), jnp.float32),
            pltpu.VMEM((ID, 2, CHUNK), jnp.int32),
            pltpu.VMEM((RD, CHUNK, dpad), jnp.float32),
            pltpu.SemaphoreType.DMA,
            pltpu.SemaphoreType.DMA,
            pltpu.SemaphoreType.DMA,
        ],
    )


@functools.cache
def _agg_kernel(dpad):
    return _make_agg(dpad, split=False)


@functools.cache
def _agg_kernel_split(dpad):
    return _make_agg(dpad, split=True)


def _row_spec(w):
    return pl.BlockSpec((_B, w), lambda i: (i, 0))


def _full_spec(shape):
    nd = len(shape)
    return pl.BlockSpec(shape, lambda i: (0,) * nd)


def _tc_matmul_split(x, wcat, splits):
    din = x.shape[1]
    wtot = wcat.shape[1]

    def body(x_ref, w_ref, *outs):
        cat = jnp.dot(x_ref[...], w_ref[...], preferred_element_type=jnp.float32)
        col = 0
        for o, w in zip(outs, splits):
            o[...] = cat[:, col:col + w]
            col += w

    return pl.pallas_call(
        body,
        grid=(_GRID,),
        in_specs=[_row_spec(din), _full_spec((din, wtot))],
        out_specs=[_row_spec(w) for w in splits],
        out_shape=[jax.ShapeDtypeStruct((N, w), jnp.float32) for w in splits],
    )(x, wcat)


def _tc_layer(a, r, s, invd, *, Wl, bl, b, ln, wcat, splits, out_h, first,
              cat_cols=False, xrs=None, np_first_split=False):
    Da = a.shape[-1]
    dop = bl.shape[-1]
    wtot = wcat.shape[1]
    ln_g, ln_b, ln_d = ln if ln is not None else (None, None, None)

    if xrs is not None:
        x_in, w_rs = xrs
        ins = [a, x_in, w_rs]
        specs = [pl.BlockSpec((2, _B, Da), lambda i: (0, i, 0)),
                 _row_spec(x_in.shape[1]), _full_spec(w_rs.shape)]
    else:
        ins = [a, r, s]
        specs = [pl.BlockSpec((2, _B, Da), lambda i: (0, i, 0)),
                 _row_spec(dop), _row_spec(dop)]
    if not first:
        ins.append(invd)
        specs.append(_row_spec(8))
    if Wl is not None:
        ins.append(Wl)
        specs.append(_full_spec(Wl.shape))
    ins += [bl, b]
    specs += [_full_spec((1, dop)), _full_spec((1, dop))]
    if ln is not None:
        ins += [ln_g, ln_b]
        specs += [_full_spec((1, dop)), _full_spec((1, dop))]
    ins.append(wcat)
    specs.append(_full_spec((dop, wtot)))

    out_shape = []
    out_specs = []
    if out_h == 'plain':
        out_shape.append(jax.ShapeDtypeStruct((NP, dop), jnp.float32))
        out_specs.append(_row_spec(dop))
    elif out_h == 'split':
        out_shape.append(jax.ShapeDtypeStruct((2, NP, 112), jnp.float32))
        out_specs.append(pl.BlockSpec((2, _B, 112), lambda i: (0, i, 0)))
    for k, w in enumerate(splits):
        rows_out = NP if (np_first_split and k == 0) else N
        out_shape.append(jax.ShapeDtypeStruct((rows_out, w), jnp.float32))
        out_specs.append(_row_spec(w))
    if first:
        out_shape.append(jax.ShapeDtypeStruct((N, 8), jnp.float32))
        out_specs.append(_row_spec(8))

    def body(*refs):
        it = iter(refs)
        a_ref = next(it)
        r_ref = next(it)
        s_ref = next(it)
        if xrs is not None:
            rs = jnp.dot(r_ref[...], s_ref[...],
                         preferred_element_type=jnp.float32)
        invd_ref = None if first else next(it)
        wl_ref = next(it) if Wl is not None else None
        bl_ref = next(it)
        b_ref = next(it)
        if ln is not None:
            lng_ref = next(it)
            lnb_ref = next(it)
        wcat_ref = next(it)
        outs = list(it)

        if cat_cols:
            asum = jnp.concatenate([a_ref[0], a_ref[1]], axis=1)
        else:
            asum = a_ref[0] + a_ref[1]
        if first:
            deg = asum[:, DEG_COL:DEG_COL + 1]
            inv = 1.0 / jnp.maximum(deg, 1.0)
        else:
            inv = invd_ref[:, :1]
        na = asum * inv
        if wl_ref is not None:
            pre = jnp.dot(na, wl_ref[...], preferred_element_type=jnp.float32)
        else:
            pre = na
        if xrs is not None:
            r_v, s_v = rs[:, :dop], rs[:, dop:]
        else:
            r_v, s_v = r_ref[...], s_ref[...]
        pre = pre + bl_ref[...] + r_v
        h = jnp.where(pre >= 0, pre, 0.01 * pre) + s_v + b_ref[...]
        if ln is not None:
            dD = float(ln_d)
            mask = lax.broadcasted_iota(jnp.int32, h.shape, 1) < ln_d
            mu = jnp.sum(h, axis=1, keepdims=True) / dD
            hc = jnp.where(mask, h - mu, 0.0)
            var = jnp.sum(hc * hc, axis=1, keepdims=True) / dD
            h = hc * lax.rsqrt(var + 1e-5) * lng_ref[...] + lnb_ref[...]
        cat = jnp.dot(h, wcat_ref[...], preferred_element_type=jnp.float32)
        k = 0
        if out_h == 'plain':
            outs[k][...] = h
            k += 1
        elif out_h == 'split':
            outs[k][0] = h[:, :112]
            outs[k][1] = jnp.concatenate(
                [h[:, 112:208], jnp.zeros((_B, 16), jnp.float32)], axis=1)
            k += 1
        col = 0
        for w in splits:
            outs[k][...] = cat[:, col:col + w]
            k += 1
            col += w
        if first:
            outs[k][...] = jnp.broadcast_to(inv, (_B, 8))

    return pl.pallas_call(
        body,
        grid=(_GRID,),
        in_specs=specs,
        out_specs=out_specs,
        out_shape=out_shape,
    )(*ins)


def _tc_final(a, r, invd, bl):
    dop = bl.shape[-1]

    def body(a_ref, r_ref, invd_ref, bl_ref, o_ref):
        asum = a_ref[0] + a_ref[1]
        o_ref[...] = asum * invd_ref[:, :1] + bl_ref[...] + r_ref[...]

    return pl.pallas_call(
        body,
        grid=(_GRID,),
        in_specs=[pl.BlockSpec((2, _B, dop), lambda i: (0, i, 0)),
                  _row_spec(dop), _row_spec(8), _full_spec((1, dop))],
        out_specs=_row_spec(dop),
        out_shape=jax.ShapeDtypeStruct((N, dop), jnp.float32),
    )(a, r, invd, bl)


def kernel(x, edge_index, params):
    f32 = jnp.float32
    ei = edge_index.reshape(2, NCHUNKS, CHUNK).transpose(1, 0, 2)

    def padw(w, rr, cc):
        return jnp.pad(w.astype(f32), ((0, rr - w.shape[0]), (0, cc - w.shape[1])))

    def padv(v, cc):
        return jnp.pad(v.astype(f32), (0, cc - v.shape[0]))[None, :]

    dins = [128, 200, 200, 100, 100, 50, 50]
    douts = [200, 200, 100, 100, 50, 50, 32]
    dinp = [_pad16(d) for d in dins]
    dop = [_pad16(d) for d in douts]

    zeros = {d: jnp.zeros((NP, d), f32) for d in (80, 112, 64, 32)}

    def agg(g):
        d = g.shape[1]
        return _agg_kernel(d)(g, ei, zeros[d])

    g0a = jnp.pad(x[:, :80], ((0, NP - N), (0, 0)))
    g0b = jnp.pad(jnp.concatenate([x[:, 80:128], jnp.ones((N, 1), f32)],
                                  axis=1), ((0, NP - N), (0, 31)))
    A0 = _agg_kernel_split(80)(g0a, g0b, ei, zeros[80])
    wl0p = padw(params["Wl0"], 128, 208)
    h1s, r1, s1, invd = _tc_layer(
        A0, None, None, None,
        xrs=(x, jnp.concatenate([padw(params["Wr0"], 128, 208),
                                 padw(params["W0"], 128, 208)], axis=1)),
        Wl=jnp.concatenate([wl0p[:80], wl0p[80:128],
                            jnp.zeros((32, 208), f32)], axis=0),
        bl=padv(params["bl0"], 208),
        b=padv(params["b0"], 208), ln=None,
        wcat=jnp.concatenate([padw(params["Wr1"], 208, 208),
                              padw(params["W1"], 208, 208)], axis=1),
        splits=[208, 208], out_h='split', first=True, cat_cols=True)

    A1 = _agg_kernel_split(112)(h1s[0], h1s[1], ei, zeros[112])
    wl1p = padw(params["Wl1"], 208, 208)
    g2, r2, s2 = _tc_layer(
        A1, r1, s1, invd,
        Wl=jnp.concatenate([wl1p[:112], wl1p[112:208],
                            jnp.zeros((16, 208), f32)], axis=0),
        bl=padv(params["bl1"], 208),
        b=padv(params["b1"], 208), ln=None,
        wcat=jnp.concatenate([padw(params["Wl2"], 208, 112),
                              padw(params["Wr2"], 208, 112),
                              padw(params["W2"], 208, 112)], axis=1),
        splits=[112, 112, 112], out_h=None, first=False, cat_cols=True,
        np_first_split=True)

    A2 = agg(g2)
    h3, r3, s3 = _tc_layer(
        A2, r2, s2, invd,
        Wl=None, bl=padv(params["bl2"], 112), b=padv(params["b2"], 112),
        ln=(padv(params["g3"], 112), padv(params["be3"], 112), 100),
        wcat=jnp.concatenate([padw(params["Wr3"], 112, 112),
                              padw(params["W3"], 112, 112)], axis=1),
        splits=[112, 112], out_h='plain', first=False)

    A3 = agg(h3)
    g4, r4, s4 = _tc_layer(
        A3, r3, s3, invd,
        Wl=padw(params["Wl3"], 112, 112), bl=padv(params["bl3"], 112),
        b=padv(params["b3"], 112), ln=None,
        wcat=jnp.concatenate([padw(params["Wl4"], 112, 64),
                              padw(params["Wr4"], 112, 64),
                              padw(params["W4"], 112, 64)], axis=1),
        splits=[64, 64, 64], out_h=None, first=False, np_first_split=True)

    A4 = agg(g4)
    h5, r5, s5 = _tc_layer(
        A4, r4, s4, invd,
        Wl=None, bl=padv(params["bl4"], 64), b=padv(params["b4"], 64),
        ln=None,
        wcat=jnp.concatenate([padw(params["Wr5"], 64, 64),
                              padw(params["W5"], 64, 64)], axis=1),
        splits=[64, 64], out_h='plain', first=False)

    A5 = agg(h5)
    g6, r6 = _tc_layer(
        A5, r5, s5, invd,
        Wl=padw(params["Wl5"], 64, 64), bl=padv(params["bl5"], 64),
        b=padv(params["b5"], 64),
        ln=(padv(params["g6"], 64), padv(params["be6"], 64), 50),
        wcat=jnp.concatenate([padw(params["Wl6"], 64, 32),
                              padw(params["Wr6"], 64, 32)], axis=1),
        splits=[32, 32], out_h=None, first=False, np_first_split=True)

    A6 = agg(g6)
    return _tc_final(A6, r6, invd, padv(params["bl6"], 32))


# speedup vs baseline: 1.0278x; 1.0278x over previous
import functools

import jax
import jax.numpy as jnp
from jax import lax
from jax.experimental import pallas as pl
from jax.experimental.pallas import tpu as pltpu
from jax.experimental.pallas import tpu_sc as plsc

N = 10000
E = 320000
NC, NS = 2, 16
NW = NC * NS
CHUNK = 128
NCHUNKS = E // CHUNK
NP = N + 16
STRIPE = 632
STRIPE_LAST = NP - STRIPE * (NS - 1)
DEG_COL = 128
RD, ID = 4, 8

_B = 2000
_GRID = N // _B


def _pad16(d):
    return -(-d // 16) * 16


def _make_agg(dpad, split):
    mesh = plsc.VectorSubcoreMesh(core_axis_name="c", subcore_axis_name="s")
    nworkers = NS if split else NW
    cnt = NCHUNKS // nworkers
    rem = NCHUNKS % nworkers

    def body(*refs):
        if split:
            (g0_hbm, g1_hbm, ei_hbm, out_hbm,
             acc, idx, rows, isem, gsem, ssem) = refs
        else:
            (g0_hbm, ei_hbm, out_hbm,
             acc, idx, rows, isem, gsem, ssem) = refs
        c = lax.axis_index("c")
        s = lax.axis_index("s")
        w = s if split else s * NC + c
        st = w * cnt + jnp.minimum(w, rem)
        row0 = s * STRIPE

        zv = jnp.zeros((16,), jnp.float32)

        def _zrow(r, carry):
            for k in range(dpad // 16):
                rows[0, r, pl.ds(k * 16, 16)] = zv
            return carry

        lax.fori_loop(0, CHUNK, _zrow, 0)
        for t in range(4):
            pltpu.sync_copy(rows.at[0],
                            acc.at[pl.ds(row0 + t * CHUNK, CHUNK)])
        pltpu.sync_copy(rows.at[0, pl.ds(0, STRIPE_LAST - 4 * CHUNK)],
                        acc.at[pl.ds(row0 + 4 * CHUNK,
                                     STRIPE_LAST - 4 * CHUNK)])

        @pl.when(s < NS - 1)
        def _zero_rest():
            pltpu.sync_copy(
                rows.at[0, pl.ds(0, STRIPE - STRIPE_LAST)],
                acc.at[pl.ds(row0 + STRIPE_LAST, STRIPE - STRIPE_LAST)])

        def run(g_hbm):
            def idx_load(j):
                pltpu.async_copy(ei_hbm.at[st + j], idx.at[j % ID], isem)

            def idx_wait(j):
                pltpu.make_async_copy(ei_hbm.at[st + j], idx.at[j % ID],
                                      isem).wait()

            def gather_start(j):
                pltpu.async_copy(g_hbm.at[idx.at[j % ID, 0]], rows.at[j % RD],
                                 gsem)

            def gather_wait(j):
                pltpu.make_async_copy(g_hbm.at[idx.at[j % ID, 0]],
                                      rows.at[j % RD], gsem).wait()

            def scat_start(j):
                pltpu.async_copy(rows.at[j % RD], acc.at[idx.at[j % ID, 1]],
                                 ssem, add=True)

            def scat_wait(j):
                pltpu.make_async_copy(rows.at[j % RD],
                                      acc.at[idx.at[j % ID, 1]], ssem).wait()

            for t in range(4):
                idx_load(t)
            idx_wait(0)
            gather_start(0)
            idx_wait(1)
            gather_start(1)
            plsc.subcore_barrier()

            for j in (0, 1):
                idx_wait(j + 2)
                gather_start(j + 2)
                idx_load(j + 4)
                gather_wait(j)
                scat_start(j)

            def step(j, carry):
                scat_wait(j - 2)
                idx_wait(j + 2)
                gather_start(j + 2)
                idx_load(j + 4)
                gather_wait(j)
                scat_start(j)
                return carry

            lax.fori_loop(2, cnt - 4, step, 0, unroll=2)

            for j in range(cnt - 4, cnt):
                scat_wait(j - 2)
                if j + 2 < cnt:
                    idx_wait(j + 2)
                    gather_start(j + 2)
                gather_wait(j)
                scat_start(j)
            scat_wait(cnt - 2)
            scat_wait(cnt - 1)

            @pl.when(w < rem)
            def _extra():
                idx_load(cnt)
                idx_wait(cnt)
                gather_start(cnt)
                gather_wait(cnt)
                scat_start(cnt)
                scat_wait(cnt)

        if split:
            @pl.when(c == 0)
            def _run0():
                run(g0_hbm)

            @pl.when(c == 1)
            def _run1():
                run(g1_hbm)
        else:
            run(g0_hbm)

        plsc.subcore_barrier()
        pltpu.sync_copy(acc.at[pl.ds(row0, STRIPE_LAST)],
                        out_hbm.at[c, pl.ds(row0, STRIPE_LAST)])

        @pl.when(s < NS - 1)
        def _out_rest():
            pltpu.sync_copy(
                acc.at[pl.ds(row0 + STRIPE_LAST, STRIPE - STRIPE_LAST)],
                out_hbm.at[c, pl.ds(row0 + STRIPE_LAST, STRIPE - STRIPE_LAST)])

    in_types = [jax.ShapeDtypeStruct((NP, dpad), jnp.float32)] * (2 if split else 1)
    del in_types

    return pl.kernel(
        body,
        out_type=jax.ShapeDtypeStruct((NC, NP, dpad), jnp.float32),
        mesh=mesh,
        compiler_params=pltpu.CompilerParams(use_tc_tiling_on_sc=False),
        scratch_types=[
            pltpu.VMEM_SHARED((NP, dpad), jnp.float32),
            pltpu.VMEM((ID, 2, CHUNK), jnp.int32),
            pltpu.VMEM((RD, CHUNK, dpad), jnp.float32),
            pltpu.SemaphoreType.DMA,
            pltpu.SemaphoreType.DMA,
            pltpu.SemaphoreType.DMA,
        ],
    )


@functools.cache
def _agg_kernel(dpad):
    return _make_agg(dpad, split=False)


@functools.cache
def _agg_kernel_split(dpad):
    return _make_agg(dpad, split=True)


def _row_spec(w):
    return pl.BlockSpec((_B, w), lambda i: (i, 0))


def _full_spec(shape):
    nd = len(shape)
    return pl.BlockSpec(shape, lambda i: (0,) * nd)


def _tc_matmul_split(x, wcat, splits):
    din = x.shape[1]
    wtot = wcat.shape[1]

    def body(x_ref, w_ref, *outs):
        cat = jnp.dot(x_ref[...], w_ref[...], preferred_element_type=jnp.float32)
        col = 0
        for o, w in zip(outs, splits):
            o[...] = cat[:, col:col + w]
            col += w

    return pl.pallas_call(
        body,
        grid=(_GRID,),
        in_specs=[_row_spec(din), _full_spec((din, wtot))],
        out_specs=[_row_spec(w) for w in splits],
        out_shape=[jax.ShapeDtypeStruct((N, w), jnp.float32) for w in splits],
    )(x, wcat)


def _tc_layer(a, r, s, invd, *, Wl, bl, b, ln, wcat, splits, out_h, first,
              cat_cols=False, xrs=None, np_first_split=False):
    Da = a.shape[-1]
    dop = bl.shape[-1]
    wtot = wcat.shape[1]
    ln_g, ln_b, ln_d = ln if ln is not None else (None, None, None)

    if xrs is not None:
        x_in, w_rs = xrs
        ins = [a, x_in, w_rs]
        specs = [pl.BlockSpec((2, _B, Da), lambda i: (0, i, 0)),
                 _row_spec(x_in.shape[1]), _full_spec(w_rs.shape)]
    else:
        ins = [a, r, s]
        specs = [pl.BlockSpec((2, _B, Da), lambda i: (0, i, 0)),
                 _row_spec(dop), _row_spec(dop)]
    if not first:
        ins.append(invd)
        specs.append(_row_spec(8))
    if Wl is not None:
        ins.append(Wl)
        specs.append(_full_spec(Wl.shape))
    ins += [bl, b]
    specs += [_full_spec((1, dop)), _full_spec((1, dop))]
    if ln is not None:
        ins += [ln_g, ln_b]
        specs += [_full_spec((1, dop)), _full_spec((1, dop))]
    ins.append(wcat)
    specs.append(_full_spec((dop, wtot)))

    out_shape = []
    out_specs = []
    if out_h == 'plain':
        out_shape.append(jax.ShapeDtypeStruct((NP, dop), jnp.float32))
        out_specs.append(_row_spec(dop))
    elif out_h == 'split':
        out_shape.append(jax.ShapeDtypeStruct((2, NP, 112), jnp.float32))
        out_specs.append(pl.BlockSpec((2, _B, 112), lambda i: (0, i, 0)))
    for k, w in enumerate(splits):
        rows_out = NP if (np_first_split and k == 0) else N
        out_shape.append(jax.ShapeDtypeStruct((rows_out, w), jnp.float32))
        out_specs.append(_row_spec(w))
    if first:
        out_shape.append(jax.ShapeDtypeStruct((N, 8), jnp.float32))
        out_specs.append(_row_spec(8))

    def body(*refs):
        it = iter(refs)
        a_ref = next(it)
        r_ref = next(it)
        s_ref = next(it)
        if xrs is not None:
            rs = jnp.dot(r_ref[...], s_ref[...],
                         preferred_element_type=jnp.float32)
        invd_ref = None if first else next(it)
        wl_ref = next(it) if Wl is not None else None
        bl_ref = next(it)
        b_ref = next(it)
        if ln is not None:
            lng_ref = next(it)
            lnb_ref = next(it)
        wcat_ref = next(it)
        outs = list(it)

        if cat_cols:
            asum = jnp.concatenate([a_ref[0], a_ref[1]], axis=1)
        else:
            asum = a_ref[0] + a_ref[1]
        if first:
            deg = asum[:, DEG_COL:DEG_COL + 1]
            inv = 1.0 / jnp.maximum(deg, 1.0)
        else:
            inv = invd_ref[:, :1]
        na = asum * inv
        if wl_ref is not None:
            pre = jnp.dot(na, wl_ref[...], preferred_element_type=jnp.float32)
        else:
            pre = na
        if xrs is not None:
            r_v, s_v = rs[:, :dop], rs[:, dop:]
        else:
            r_v, s_v = r_ref[...], s_ref[...]
        pre = pre + bl_ref[...] + r_v
        h = jnp.where(pre >= 0, pre, 0.01 * pre) + s_v + b_ref[...]
        if ln is not None:
            dD = float(ln_d)
            mask = lax.broadcasted_iota(jnp.int32, h.shape, 1) < ln_d
            mu = jnp.sum(h, axis=1, keepdims=True) / dD
            hc = jnp.where(mask, h - mu, 0.0)
            var = jnp.sum(hc * hc, axis=1, keepdims=True) / dD
            h = hc * lax.rsqrt(var + 1e-5) * lng_ref[...] + lnb_ref[...]
        cat = jnp.dot(h, wcat_ref[...], preferred_element_type=jnp.float32)
        k = 0
        if out_h == 'plain':
            outs[k][...] = h
            k += 1
        elif out_h == 'split':
            outs[k][0] = h[:, :112]
            outs[k][1] = jnp.concatenate(
                [h[:, 112:208], jnp.zeros((_B, 16), jnp.float32)], axis=1)
            k += 1
        col = 0
        for w in splits:
            outs[k][...] = cat[:, col:col + w]
            k += 1
            col += w
        if first:
            outs[k][...] = jnp.broadcast_to(inv, (_B, 8))

    return pl.pallas_call(
        body,
        grid=(_GRID,),
        in_specs=specs,
        out_specs=out_specs,
        out_shape=out_shape,
    )(*ins)


def _tc_final(a, r, invd, bl):
    dop = bl.shape[-1]

    def body(a_ref, r_ref, invd_ref, bl_ref, o_ref):
        asum = a_ref[0] + a_ref[1]
        o_ref[...] = asum * invd_ref[:, :1] + bl_ref[...] + r_ref[...]

    return pl.pallas_call(
        body,
        grid=(_GRID,),
        in_specs=[pl.BlockSpec((2, _B, dop), lambda i: (0, i, 0)),
                  _row_spec(dop), _row_spec(8), _full_spec((1, dop))],
        out_specs=_row_spec(dop),
        out_shape=jax.ShapeDtypeStruct((N, dop), jnp.float32),
    )(a, r, invd, bl)


def kernel(x, edge_index, params):
    f32 = jnp.float32
    ei = edge_index.reshape(2, NCHUNKS, CHUNK).transpose(1, 0, 2)

    def padw(w, rr, cc):
        return jnp.pad(w.astype(f32), ((0, rr - w.shape[0]), (0, cc - w.shape[1])))

    def padv(v, cc):
        return jnp.pad(v.astype(f32), (0, cc - v.shape[0]))[None, :]

    dins = [128, 200, 200, 100, 100, 50, 50]
    douts = [200, 200, 100, 100, 50, 50, 32]
    dinp = [_pad16(d) for d in dins]
    dop = [_pad16(d) for d in douts]

    def agg(g):
        return _agg_kernel(g.shape[1])(g, ei)

    g0a = jnp.pad(x[:, :80], ((0, NP - N), (0, 0)))
    g0b = jnp.pad(jnp.concatenate([x[:, 80:128], jnp.ones((N, 1), f32)],
                                  axis=1), ((0, NP - N), (0, 31)))
    A0 = _agg_kernel_split(80)(g0a, g0b, ei)
    wl0p = padw(params["Wl0"], 128, 208)
    h1s, r1, s1, invd = _tc_layer(
        A0, None, None, None,
        xrs=(x, jnp.concatenate([padw(params["Wr0"], 128, 208),
                                 padw(params["W0"], 128, 208)], axis=1)),
        Wl=jnp.concatenate([wl0p[:80], wl0p[80:128],
                            jnp.zeros((32, 208), f32)], axis=0),
        bl=padv(params["bl0"], 208),
        b=padv(params["b0"], 208), ln=None,
        wcat=jnp.concatenate([padw(params["Wr1"], 208, 208),
                              padw(params["W1"], 208, 208)], axis=1),
        splits=[208, 208], out_h='split', first=True, cat_cols=True)

    A1 = _agg_kernel_split(112)(h1s[0], h1s[1], ei)
    wl1p = padw(params["Wl1"], 208, 208)
    g2, r2, s2 = _tc_layer(
        A1, r1, s1, invd,
        Wl=jnp.concatenate([wl1p[:112], wl1p[112:208],
                            jnp.zeros((16, 208), f32)], axis=0),
        bl=padv(params["bl1"], 208),
        b=padv(params["b1"], 208), ln=None,
        wcat=jnp.concatenate([padw(params["Wl2"], 208, 112),
                              padw(params["Wr2"], 208, 112),
                              padw(params["W2"], 208, 112)], axis=1),
        splits=[112, 112, 112], out_h=None, first=False, cat_cols=True,
        np_first_split=True)

    A2 = agg(g2)
    h3, r3, s3 = _tc_layer(
        A2, r2, s2, invd,
        Wl=None, bl=padv(params["bl2"], 112), b=padv(params["b2"], 112),
        ln=(padv(params["g3"], 112), padv(params["be3"], 112), 100),
        wcat=jnp.concatenate([padw(params["Wr3"], 112, 112),
                              padw(params["W3"], 112, 112)], axis=1),
        splits=[112, 112], out_h='plain', first=False)

    A3 = agg(h3)
    g4, r4, s4 = _tc_layer(
        A3, r3, s3, invd,
        Wl=padw(params["Wl3"], 112, 112), bl=padv(params["bl3"], 112),
        b=padv(params["b3"], 112), ln=None,
        wcat=jnp.concatenate([padw(params["Wl4"], 112, 64),
                              padw(params["Wr4"], 112, 64),
                              padw(params["W4"], 112, 64)], axis=1),
        splits=[64, 64, 64], out_h=None, first=False, np_first_split=True)

    A4 = agg(g4)
    h5, r5, s5 = _tc_layer(
        A4, r4, s4, invd,
        Wl=None, bl=padv(params["bl4"], 64), b=padv(params["b4"], 64),
        ln=None,
        wcat=jnp.concatenate([padw(params["Wr5"], 64, 64),
                              padw(params["W5"], 64, 64)], axis=1),
        splits=[64, 64], out_h='plain', first=False)

    A5 = agg(h5)
    g6, r6 = _tc_layer(
        A5, r5, s5, invd,
        Wl=padw(params["Wl5"], 64, 64), bl=padv(params["bl5"], 64),
        b=padv(params["b5"], 64),
        ln=(padv(params["g6"], 64), padv(params["be6"], 64), 50),
        wcat=jnp.concatenate([padw(params["Wl6"], 64, 32),
                              padw(params["Wr6"], 64, 32)], axis=1),
        splits=[32, 32], out_h=None, first=False, np_first_split=True)

    A6 = agg(g6)
    return _tc_final(A6, r6, invd, padv(params["bl6"], 32))


# speedup vs baseline: 1.0324x; 1.0045x over previous
import functools

import jax
import jax.numpy as jnp
from jax import lax
from jax.experimental import pallas as pl
from jax.experimental.pallas import tpu as pltpu
from jax.experimental.pallas import tpu_sc as plsc

N = 10000
E = 320000
NC, NS = 2, 16
NW = NC * NS
CHUNK = 128
NCHUNKS = E // CHUNK
NP = N + 16
STRIPE = 632
STRIPE_LAST = NP - STRIPE * (NS - 1)
DEG_COL = 128
RD, ID = 4, 8

_B = 2000
_GRID = N // _B


def _pad16(d):
    return -(-d // 16) * 16


def _make_agg(dpad, split):
    mesh = plsc.VectorSubcoreMesh(core_axis_name="c", subcore_axis_name="s")
    nworkers = NS if split else NW
    cnt = NCHUNKS // nworkers
    rem = NCHUNKS % nworkers

    def body(*refs):
        if split:
            (g0_hbm, g1_hbm, ei_hbm, out_hbm,
             acc, idx, rows, isem, gsem, ssem) = refs
        else:
            (g0_hbm, ei_hbm, out_hbm,
             acc, idx, rows, isem, gsem, ssem) = refs
        c = lax.axis_index("c")
        s = lax.axis_index("s")
        w = s if split else s * NC + c
        st = w * cnt + jnp.minimum(w, rem)
        row0 = s * STRIPE

        def idx_load(j):
            pltpu.async_copy(ei_hbm.at[st + j], idx.at[j % ID], isem)

        def idx_wait(j):
            pltpu.make_async_copy(ei_hbm.at[st + j], idx.at[j % ID],
                                  isem).wait()

        for t in range(4):
            idx_load(t)

        zv = jnp.zeros((16,), jnp.float32)

        def _zrow(r, carry):
            for k in range(dpad // 16):
                rows[0, r, pl.ds(k * 16, 16)] = zv
            return carry

        lax.fori_loop(0, CHUNK, _zrow, 0)
        for t in range(4):
            pltpu.sync_copy(rows.at[0],
                            acc.at[pl.ds(row0 + t * CHUNK, CHUNK)])
        pltpu.sync_copy(rows.at[0, pl.ds(0, STRIPE_LAST - 4 * CHUNK)],
                        acc.at[pl.ds(row0 + 4 * CHUNK,
                                     STRIPE_LAST - 4 * CHUNK)])

        @pl.when(s < NS - 1)
        def _zero_rest():
            pltpu.sync_copy(
                rows.at[0, pl.ds(0, STRIPE - STRIPE_LAST)],
                acc.at[pl.ds(row0 + STRIPE_LAST, STRIPE - STRIPE_LAST)])

        def run(g_hbm):
            def gather_start(j):
                pltpu.async_copy(g_hbm.at[idx.at[j % ID, 0]], rows.at[j % RD],
                                 gsem)

            def gather_wait(j):
                pltpu.make_async_copy(g_hbm.at[idx.at[j % ID, 0]],
                                      rows.at[j % RD], gsem).wait()

            def scat_start(j):
                pltpu.async_copy(rows.at[j % RD], acc.at[idx.at[j % ID, 1]],
                                 ssem, add=True)

            def scat_wait(j):
                pltpu.make_async_copy(rows.at[j % RD],
                                      acc.at[idx.at[j % ID, 1]], ssem).wait()

            idx_wait(0)
            gather_start(0)
            idx_wait(1)
            gather_start(1)
            plsc.subcore_barrier()

            for j in (0, 1):
                idx_wait(j + 2)
                gather_start(j + 2)
                idx_load(j + 4)
                gather_wait(j)
                scat_start(j)

            def step(j, carry):
                scat_wait(j - 2)
                idx_wait(j + 2)
                gather_start(j + 2)
                idx_load(j + 4)
                gather_wait(j)
                scat_start(j)
                return carry

            lax.fori_loop(2, cnt - 4, step, 0, unroll=2)

            for j in range(cnt - 4, cnt):
                scat_wait(j - 2)
                if j + 2 < cnt:
                    idx_wait(j + 2)
                    gather_start(j + 2)
                gather_wait(j)
                scat_start(j)
            scat_wait(cnt - 2)
            scat_wait(cnt - 1)

            @pl.when(w < rem)
            def _extra():
                idx_load(cnt)
                idx_wait(cnt)
                gather_start(cnt)
                gather_wait(cnt)
                scat_start(cnt)
                scat_wait(cnt)

        if split:
            @pl.when(c == 0)
            def _run0():
                run(g0_hbm)

            @pl.when(c == 1)
            def _run1():
                run(g1_hbm)
        else:
            run(g0_hbm)

        plsc.subcore_barrier()
        pltpu.sync_copy(acc.at[pl.ds(row0, STRIPE_LAST)],
                        out_hbm.at[c, pl.ds(row0, STRIPE_LAST)])

        @pl.when(s < NS - 1)
        def _out_rest():
            pltpu.sync_copy(
                acc.at[pl.ds(row0 + STRIPE_LAST, STRIPE - STRIPE_LAST)],
                out_hbm.at[c, pl.ds(row0 + STRIPE_LAST, STRIPE - STRIPE_LAST)])

    in_types = [jax.ShapeDtypeStruct((NP, dpad), jnp.float32)] * (2 if split else 1)
    del in_types

    return pl.kernel(
        body,
        out_type=jax.ShapeDtypeStruct((NC, NP, dpad), jnp.float32),
        mesh=mesh,
        compiler_params=pltpu.CompilerParams(use_tc_tiling_on_sc=False),
        scratch_types=[
            pltpu.VMEM_SHARED((NP, dpad), jnp.float32),
            pltpu.VMEM((ID, 2, CHUNK), jnp.int32),
            pltpu.VMEM((RD, CHUNK, dpad), jnp.float32),
            pltpu.SemaphoreType.DMA,
            pltpu.SemaphoreType.DMA,
            pltpu.SemaphoreType.DMA,
        ],
    )


@functools.cache
def _agg_kernel(dpad):
    return _make_agg(dpad, split=False)


@functools.cache
def _agg_kernel_split(dpad):
    return _make_agg(dpad, split=True)


def _row_spec(w):
    return pl.BlockSpec((_B, w), lambda i: (i, 0))


def _full_spec(shape):
    nd = len(shape)
    return pl.BlockSpec(shape, lambda i: (0,) * nd)


def _tc_matmul_split(x, wcat, splits):
    din = x.shape[1]
    wtot = wcat.shape[1]

    def body(x_ref, w_ref, *outs):
        cat = jnp.dot(x_ref[...], w_ref[...], preferred_element_type=jnp.float32)
        col = 0
        for o, w in zip(outs, splits):
            o[...] = cat[:, col:col + w]
            col += w

    return pl.pallas_call(
        body,
        grid=(_GRID,),
        in_specs=[_row_spec(din), _full_spec((din, wtot))],
        out_specs=[_row_spec(w) for w in splits],
        out_shape=[jax.ShapeDtypeStruct((N, w), jnp.float32) for w in splits],
    )(x, wcat)


def _tc_layer(a, r, s, invd, *, Wl, bl, b, ln, wcat, splits, out_h, first,
              cat_cols=False, xrs=None, np_first_split=False):
    Da = a.shape[-1]
    dop = bl.shape[-1]
    wtot = wcat.shape[1]
    ln_g, ln_b, ln_d = ln if ln is not None else (None, None, None)

    if xrs is not None:
        x_in, w_rs = xrs
        ins = [a, x_in, w_rs]
        specs = [pl.BlockSpec((2, _B, Da), lambda i: (0, i, 0)),
                 _row_spec(x_in.shape[1]), _full_spec(w_rs.shape)]
    else:
        ins = [a, r, s]
        specs = [pl.BlockSpec((2, _B, Da), lambda i: (0, i, 0)),
                 _row_spec(dop), _row_spec(dop)]
    if not first:
        ins.append(invd)
        specs.append(_row_spec(8))
    if Wl is not None:
        ins.append(Wl)
        specs.append(_full_spec(Wl.shape))
    ins += [bl, b]
    specs += [_full_spec((1, dop)), _full_spec((1, dop))]
    if ln is not None:
        ins += [ln_g, ln_b]
        specs += [_full_spec((1, dop)), _full_spec((1, dop))]
    ins.append(wcat)
    specs.append(_full_spec((dop, wtot)))

    out_shape = []
    out_specs = []
    if out_h == 'plain':
        out_shape.append(jax.ShapeDtypeStruct((NP, dop), jnp.float32))
        out_specs.append(_row_spec(dop))
    elif out_h == 'split':
        out_shape.append(jax.ShapeDtypeStruct((2, NP, 112), jnp.float32))
        out_specs.append(pl.BlockSpec((2, _B, 112), lambda i: (0, i, 0)))
    for k, w in enumerate(splits):
        rows_out = NP if (np_first_split and k == 0) else N
        out_shape.append(jax.ShapeDtypeStruct((rows_out, w), jnp.float32))
        out_specs.append(_row_spec(w))
    if first:
        out_shape.append(jax.ShapeDtypeStruct((N, 8), jnp.float32))
        out_specs.append(_row_spec(8))

    def body(*refs):
        it = iter(refs)
        a_ref = next(it)
        r_ref = next(it)
        s_ref = next(it)
        if xrs is not None:
            rs = jnp.dot(r_ref[...], s_ref[...],
                         preferred_element_type=jnp.float32)
        invd_ref = None if first else next(it)
        wl_ref = next(it) if Wl is not None else None
        bl_ref = next(it)
        b_ref = next(it)
        if ln is not None:
            lng_ref = next(it)
            lnb_ref = next(it)
        wcat_ref = next(it)
        outs = list(it)

        if cat_cols:
            asum = jnp.concatenate([a_ref[0], a_ref[1]], axis=1)
        else:
            asum = a_ref[0] + a_ref[1]
        if first:
            deg = asum[:, DEG_COL:DEG_COL + 1]
            inv = 1.0 / jnp.maximum(deg, 1.0)
        else:
            inv = invd_ref[:, :1]
        na = asum * inv
        if wl_ref is not None:
            pre = jnp.dot(na, wl_ref[...], preferred_element_type=jnp.float32)
        else:
            pre = na
        if xrs is not None:
            r_v, s_v = rs[:, :dop], rs[:, dop:]
        else:
            r_v, s_v = r_ref[...], s_ref[...]
        pre = pre + bl_ref[...] + r_v
        h = jnp.where(pre >= 0, pre, 0.01 * pre) + s_v + b_ref[...]
        if ln is not None:
            dD = float(ln_d)
            mask = lax.broadcasted_iota(jnp.int32, h.shape, 1) < ln_d
            mu = jnp.sum(h, axis=1, keepdims=True) / dD
            hc = jnp.where(mask, h - mu, 0.0)
            var = jnp.sum(hc * hc, axis=1, keepdims=True) / dD
            h = hc * lax.rsqrt(var + 1e-5) * lng_ref[...] + lnb_ref[...]
        cat = jnp.dot(h, wcat_ref[...], preferred_element_type=jnp.float32)
        k = 0
        if out_h == 'plain':
            outs[k][...] = h
            k += 1
        elif out_h == 'split':
            outs[k][0] = h[:, :112]
            outs[k][1] = jnp.concatenate(
                [h[:, 112:208], jnp.zeros((_B, 16), jnp.float32)], axis=1)
            k += 1
        col = 0
        for w in splits:
            outs[k][...] = cat[:, col:col + w]
            k += 1
            col += w
        if first:
            outs[k][...] = jnp.broadcast_to(inv, (_B, 8))

    return pl.pallas_call(
        body,
        grid=(_GRID,),
        in_specs=specs,
        out_specs=out_specs,
        out_shape=out_shape,
    )(*ins)


def _tc_final(a, r, invd, bl):
    dop = bl.shape[-1]

    def body(a_ref, r_ref, invd_ref, bl_ref, o_ref):
        asum = a_ref[0] + a_ref[1]
        o_ref[...] = asum * invd_ref[:, :1] + bl_ref[...] + r_ref[...]

    return pl.pallas_call(
        body,
        grid=(_GRID,),
        in_specs=[pl.BlockSpec((2, _B, dop), lambda i: (0, i, 0)),
                  _row_spec(dop), _row_spec(8), _full_spec((1, dop))],
        out_specs=_row_spec(dop),
        out_shape=jax.ShapeDtypeStruct((N, dop), jnp.float32),
    )(a, r, invd, bl)


def kernel(x, edge_index, params):
    f32 = jnp.float32
    ei = edge_index.reshape(2, NCHUNKS, CHUNK).transpose(1, 0, 2)

    def padw(w, rr, cc):
        return jnp.pad(w.astype(f32), ((0, rr - w.shape[0]), (0, cc - w.shape[1])))

    def padv(v, cc):
        return jnp.pad(v.astype(f32), (0, cc - v.shape[0]))[None, :]

    dins = [128, 200, 200, 100, 100, 50, 50]
    douts = [200, 200, 100, 100, 50, 50, 32]
    dinp = [_pad16(d) for d in dins]
    dop = [_pad16(d) for d in douts]

    def agg(g):
        return _agg_kernel(g.shape[1])(g, ei)

    g0a = jnp.pad(x[:, :80], ((0, NP - N), (0, 0)))
    g0b = jnp.pad(jnp.concatenate([x[:, 80:128], jnp.ones((N, 1), f32)],
                                  axis=1), ((0, NP - N), (0, 31)))
    A0 = _agg_kernel_split(80)(g0a, g0b, ei)
    wl0p = padw(params["Wl0"], 128, 208)
    h1s, r1, s1, invd = _tc_layer(
        A0, None, None, None,
        xrs=(x, jnp.concatenate([padw(params["Wr0"], 128, 208),
                                 padw(params["W0"], 128, 208)], axis=1)),
        Wl=jnp.concatenate([wl0p[:80], wl0p[80:128],
                            jnp.zeros((32, 208), f32)], axis=0),
        bl=padv(params["bl0"], 208),
        b=padv(params["b0"], 208), ln=None,
        wcat=jnp.concatenate([padw(params["Wr1"], 208, 208),
                              padw(params["W1"], 208, 208)], axis=1),
        splits=[208, 208], out_h='split', first=True, cat_cols=True)

    A1 = _agg_kernel_split(112)(h1s[0], h1s[1], ei)
    wl1p = padw(params["Wl1"], 208, 208)
    g2, r2, s2 = _tc_layer(
        A1, r1, s1, invd,
        Wl=jnp.concatenate([wl1p[:112], wl1p[112:208],
                            jnp.zeros((16, 208), f32)], axis=0),
        bl=padv(params["bl1"], 208),
        b=padv(params["b1"], 208), ln=None,
        wcat=jnp.concatenate([padw(params["Wl2"], 208, 112),
                              padw(params["Wr2"], 208, 112),
                              padw(params["W2"], 208, 112)], axis=1),
        splits=[112, 112, 112], out_h=None, first=False, cat_cols=True,
        np_first_split=True)

    A2 = agg(g2)
    h3, r3, s3 = _tc_layer(
        A2, r2, s2, invd,
        Wl=None, bl=padv(params["bl2"], 112), b=padv(params["b2"], 112),
        ln=(padv(params["g3"], 112), padv(params["be3"], 112), 100),
        wcat=jnp.concatenate([padw(params["Wr3"], 112, 112),
                              padw(params["W3"], 112, 112)], axis=1),
        splits=[112, 112], out_h='plain', first=False)

    A3 = agg(h3)
    g4, r4, s4 = _tc_layer(
        A3, r3, s3, invd,
        Wl=padw(params["Wl3"], 112, 112), bl=padv(params["bl3"], 112),
        b=padv(params["b3"], 112), ln=None,
        wcat=jnp.concatenate([padw(params["Wl4"], 112, 64),
                              padw(params["Wr4"], 112, 64),
                              padw(params["W4"], 112, 64)], axis=1),
        splits=[64, 64, 64], out_h=None, first=False, np_first_split=True)

    A4 = agg(g4)
    h5, r5, s5 = _tc_layer(
        A4, r4, s4, invd,
        Wl=None, bl=padv(params["bl4"], 64), b=padv(params["b4"], 64),
        ln=None,
        wcat=jnp.concatenate([padw(params["Wr5"], 64, 64),
                              padw(params["W5"], 64, 64)], axis=1),
        splits=[64, 64], out_h='plain', first=False)

    A5 = agg(h5)
    g6, r6 = _tc_layer(
        A5, r5, s5, invd,
        Wl=padw(params["Wl5"], 64, 64), bl=padv(params["bl5"], 64),
        b=padv(params["b5"], 64),
        ln=(padv(params["g6"], 64), padv(params["be6"], 64), 50),
        wcat=jnp.concatenate([padw(params["Wl6"], 64, 32),
                              padw(params["Wr6"], 64, 32)], axis=1),
        splits=[32, 32], out_h=None, first=False, np_first_split=True)

    A6 = agg(g6)
    return _tc_final(A6, r6, invd, padv(params["bl6"], 32))


# speedup vs baseline: 1.0444x; 1.0117x over previous
import functools

import jax
import jax.numpy as jnp
from jax import lax
from jax.experimental import pallas as pl
from jax.experimental.pallas import tpu as pltpu
from jax.experimental.pallas import tpu_sc as plsc

N = 10000
E = 320000
NC, NS = 2, 16
NW = NC * NS
CHUNK = 128
NCHUNKS = E // CHUNK
NP = N + 16
STRIPE = 632
STRIPE_LAST = NP - STRIPE * (NS - 1)
DEG_COL = 128
RD, ID = 4, 8

_B = 2000
_GRID = N // _B


def _pad16(d):
    return -(-d // 16) * 16


def _make_agg(dpad, split):
    mesh = plsc.VectorSubcoreMesh(core_axis_name="c", subcore_axis_name="s")
    nworkers = NS if split else NW
    cnt = NCHUNKS // nworkers
    rem = NCHUNKS % nworkers

    def body(*refs):
        if split:
            (g0_hbm, g1_hbm, ei_hbm, out_hbm,
             acc, idx, rows, isem, gsem, ssem) = refs
        else:
            (g0_hbm, ei_hbm, out_hbm,
             acc, idx, rows, isem, gsem, ssem) = refs
        c = lax.axis_index("c")
        s = lax.axis_index("s")
        w = s if split else s * NC + c
        st = w * cnt + jnp.minimum(w, rem)
        row0 = s * STRIPE

        def idx_load(j):
            pltpu.async_copy(ei_hbm.at[st + j], idx.at[j % ID], isem)

        def idx_wait(j):
            pltpu.make_async_copy(ei_hbm.at[st + j], idx.at[j % ID],
                                  isem).wait()

        for t in range(4):
            idx_load(t)

        zv = jnp.zeros((16,), jnp.float32)

        def _zrow(r, carry):
            for k in range(dpad // 16):
                rows[0, r, pl.ds(k * 16, 16)] = zv
            return carry

        lax.fori_loop(0, CHUNK, _zrow, 0)
        for t in range(4):
            pltpu.sync_copy(rows.at[0],
                            acc.at[pl.ds(row0 + t * CHUNK, CHUNK)])
        pltpu.sync_copy(rows.at[0, pl.ds(0, STRIPE_LAST - 4 * CHUNK)],
                        acc.at[pl.ds(row0 + 4 * CHUNK,
                                     STRIPE_LAST - 4 * CHUNK)])

        @pl.when(s < NS - 1)
        def _zero_rest():
            pltpu.sync_copy(
                rows.at[0, pl.ds(0, STRIPE - STRIPE_LAST)],
                acc.at[pl.ds(row0 + STRIPE_LAST, STRIPE - STRIPE_LAST)])

        def run(g_hbm):
            def gather_start(j):
                pltpu.async_copy(g_hbm.at[idx.at[j % ID, 0]], rows.at[j % RD],
                                 gsem)

            def gather_wait(j):
                pltpu.make_async_copy(g_hbm.at[idx.at[j % ID, 0]],
                                      rows.at[j % RD], gsem).wait()

            def scat_start(j):
                pltpu.async_copy(rows.at[j % RD], acc.at[idx.at[j % ID, 1]],
                                 ssem, add=True)

            def scat_wait(j):
                pltpu.make_async_copy(rows.at[j % RD],
                                      acc.at[idx.at[j % ID, 1]], ssem).wait()

            idx_wait(0)
            gather_start(0)
            idx_wait(1)
            gather_start(1)
            plsc.subcore_barrier()

            for j in (0, 1):
                idx_wait(j + 2)
                gather_start(j + 2)
                idx_load(j + 4)
                gather_wait(j)
                scat_start(j)

            def step(j, carry):
                scat_wait(j - 2)
                idx_wait(j + 2)
                gather_start(j + 2)
                idx_load(j + 4)
                gather_wait(j)
                scat_start(j)
                return carry

            lax.fori_loop(2, cnt - 4, step, 0, unroll=2)

            @pl.when(w < rem)
            def _extra_idx():
                idx_load(cnt)

            for j in range(cnt - 4, cnt):
                scat_wait(j - 2)
                if j == cnt - 2:
                    @pl.when(w < rem)
                    def _extra_gather():
                        idx_wait(cnt)
                        gather_start(cnt)
                if j + 2 < cnt:
                    idx_wait(j + 2)
                    gather_start(j + 2)
                gather_wait(j)
                scat_start(j)
            scat_wait(cnt - 2)
            scat_wait(cnt - 1)

            @pl.when(w < rem)
            def _extra():
                gather_wait(cnt)
                scat_start(cnt)
                scat_wait(cnt)

        if split:
            @pl.when(c == 0)
            def _run0():
                run(g0_hbm)

            @pl.when(c == 1)
            def _run1():
                run(g1_hbm)
        else:
            run(g0_hbm)

        plsc.subcore_barrier()
        pltpu.sync_copy(acc.at[pl.ds(row0, STRIPE_LAST)],
                        out_hbm.at[c, pl.ds(row0, STRIPE_LAST)])

        @pl.when(s < NS - 1)
        def _out_rest():
            pltpu.sync_copy(
                acc.at[pl.ds(row0 + STRIPE_LAST, STRIPE - STRIPE_LAST)],
                out_hbm.at[c, pl.ds(row0 + STRIPE_LAST, STRIPE - STRIPE_LAST)])

    in_types = [jax.ShapeDtypeStruct((NP, dpad), jnp.float32)] * (2 if split else 1)
    del in_types

    return pl.kernel(
        body,
        out_type=jax.ShapeDtypeStruct((NC, NP, dpad), jnp.float32),
        mesh=mesh,
        compiler_params=pltpu.CompilerParams(use_tc_tiling_on_sc=False),
        scratch_types=[
            pltpu.VMEM_SHARED((NP, dpad), jnp.float32),
            pltpu.VMEM((ID, 2, CHUNK), jnp.int32),
            pltpu.VMEM((RD, CHUNK, dpad), jnp.float32),
            pltpu.SemaphoreType.DMA,
            pltpu.SemaphoreType.DMA,
            pltpu.SemaphoreType.DMA,
        ],
    )


@functools.cache
def _agg_kernel(dpad):
    return _make_agg(dpad, split=False)


@functools.cache
def _agg_kernel_split(dpad):
    return _make_agg(dpad, split=True)


def _row_spec(w):
    return pl.BlockSpec((_B, w), lambda i: (i, 0))


def _full_spec(shape):
    nd = len(shape)
    return pl.BlockSpec(shape, lambda i: (0,) * nd)


def _tc_matmul_split(x, wcat, splits):
    din = x.shape[1]
    wtot = wcat.shape[1]

    def body(x_ref, w_ref, *outs):
        cat = jnp.dot(x_ref[...], w_ref[...], preferred_element_type=jnp.float32)
        col = 0
        for o, w in zip(outs, splits):
            o[...] = cat[:, col:col + w]
            col += w

    return pl.pallas_call(
        body,
        grid=(_GRID,),
        in_specs=[_row_spec(din), _full_spec((din, wtot))],
        out_specs=[_row_spec(w) for w in splits],
        out_shape=[jax.ShapeDtypeStruct((N, w), jnp.float32) for w in splits],
    )(x, wcat)


def _tc_layer(a, r, s, invd, *, Wl, bl, b, ln, wcat, splits, out_h, first,
              cat_cols=False, xrs=None, np_first_split=False):
    Da = a.shape[-1]
    dop = bl.shape[-1]
    wtot = wcat.shape[1]
    ln_g, ln_b, ln_d = ln if ln is not None else (None, None, None)

    if xrs is not None:
        x_in, w_rs = xrs
        ins = [a, x_in, w_rs]
        specs = [pl.BlockSpec((2, _B, Da), lambda i: (0, i, 0)),
                 _row_spec(x_in.shape[1]), _full_spec(w_rs.shape)]
    else:
        ins = [a, r, s]
        specs = [pl.BlockSpec((2, _B, Da), lambda i: (0, i, 0)),
                 _row_spec(dop), _row_spec(dop)]
    if not first:
        ins.append(invd)
        specs.append(_row_spec(8))
    if Wl is not None:
        ins.append(Wl)
        specs.append(_full_spec(Wl.shape))
    ins += [bl, b]
    specs += [_full_spec((1, dop)), _full_spec((1, dop))]
    if ln is not None:
        ins += [ln_g, ln_b]
        specs += [_full_spec((1, dop)), _full_spec((1, dop))]
    ins.append(wcat)
    specs.append(_full_spec((dop, wtot)))

    out_shape = []
    out_specs = []
    if out_h == 'plain':
        out_shape.append(jax.ShapeDtypeStruct((NP, dop), jnp.float32))
        out_specs.append(_row_spec(dop))
    elif out_h == 'split':
        out_shape.append(jax.ShapeDtypeStruct((2, NP, 112), jnp.float32))
        out_specs.append(pl.BlockSpec((2, _B, 112), lambda i: (0, i, 0)))
    for k, w in enumerate(splits):
        rows_out = NP if (np_first_split and k == 0) else N
        out_shape.append(jax.ShapeDtypeStruct((rows_out, w), jnp.float32))
        out_specs.append(_row_spec(w))
    if first:
        out_shape.append(jax.ShapeDtypeStruct((N, 8), jnp.float32))
        out_specs.append(_row_spec(8))

    def body(*refs):
        it = iter(refs)
        a_ref = next(it)
        r_ref = next(it)
        s_ref = next(it)
        if xrs is not None:
            rs = jnp.dot(r_ref[...], s_ref[...],
                         preferred_element_type=jnp.float32)
        invd_ref = None if first else next(it)
        wl_ref = next(it) if Wl is not None else None
        bl_ref = next(it)
        b_ref = next(it)
        if ln is not None:
            lng_ref = next(it)
            lnb_ref = next(it)
        wcat_ref = next(it)
        outs = list(it)

        if cat_cols:
            asum = jnp.concatenate([a_ref[0], a_ref[1]], axis=1)
        else:
            asum = a_ref[0] + a_ref[1]
        if first:
            deg = asum[:, DEG_COL:DEG_COL + 1]
            inv = 1.0 / jnp.maximum(deg, 1.0)
        else:
            inv = invd_ref[:, :1]
        na = asum * inv
        if wl_ref is not None:
            pre = jnp.dot(na, wl_ref[...], preferred_element_type=jnp.float32)
        else:
            pre = na
        if xrs is not None:
            r_v, s_v = rs[:, :dop], rs[:, dop:]
        else:
            r_v, s_v = r_ref[...], s_ref[...]
        pre = pre + bl_ref[...] + r_v
        h = jnp.where(pre >= 0, pre, 0.01 * pre) + s_v + b_ref[...]
        if ln is not None:
            dD = float(ln_d)
            mask = lax.broadcasted_iota(jnp.int32, h.shape, 1) < ln_d
            mu = jnp.sum(h, axis=1, keepdims=True) / dD
            hc = jnp.where(mask, h - mu, 0.0)
            var = jnp.sum(hc * hc, axis=1, keepdims=True) / dD
            h = hc * lax.rsqrt(var + 1e-5) * lng_ref[...] + lnb_ref[...]
        cat = jnp.dot(h, wcat_ref[...], preferred_element_type=jnp.float32)
        k = 0
        if out_h == 'plain':
            outs[k][...] = h
            k += 1
        elif out_h == 'split':
            outs[k][0] = h[:, :112]
            outs[k][1] = jnp.concatenate(
                [h[:, 112:208], jnp.zeros((_B, 16), jnp.float32)], axis=1)
            k += 1
        col = 0
        for w in splits:
            outs[k][...] = cat[:, col:col + w]
            k += 1
            col += w
        if first:
            outs[k][...] = jnp.broadcast_to(inv, (_B, 8))

    return pl.pallas_call(
        body,
        grid=(_GRID,),
        in_specs=specs,
        out_specs=out_specs,
        out_shape=out_shape,
    )(*ins)


def _tc_final(a, r, invd, bl):
    dop = bl.shape[-1]

    def body(a_ref, r_ref, invd_ref, bl_ref, o_ref):
        asum = a_ref[0] + a_ref[1]
        o_ref[...] = asum * invd_ref[:, :1] + bl_ref[...] + r_ref[...]

    return pl.pallas_call(
        body,
        grid=(_GRID,),
        in_specs=[pl.BlockSpec((2, _B, dop), lambda i: (0, i, 0)),
                  _row_spec(dop), _row_spec(8), _full_spec((1, dop))],
        out_specs=_row_spec(dop),
        out_shape=jax.ShapeDtypeStruct((N, dop), jnp.float32),
    )(a, r, invd, bl)


def kernel(x, edge_index, params):
    f32 = jnp.float32
    ei = edge_index.reshape(2, NCHUNKS, CHUNK).transpose(1, 0, 2)

    def padw(w, rr, cc):
        return jnp.pad(w.astype(f32), ((0, rr - w.shape[0]), (0, cc - w.shape[1])))

    def padv(v, cc):
        return jnp.pad(v.astype(f32), (0, cc - v.shape[0]))[None, :]

    dins = [128, 200, 200, 100, 100, 50, 50]
    douts = [200, 200, 100, 100, 50, 50, 32]
    dinp = [_pad16(d) for d in dins]
    dop = [_pad16(d) for d in douts]

    def agg(g):
        return _agg_kernel(g.shape[1])(g, ei)

    g0a = jnp.pad(x[:, :80], ((0, NP - N), (0, 0)))
    g0b = jnp.pad(jnp.concatenate([x[:, 80:128], jnp.ones((N, 1), f32)],
                                  axis=1), ((0, NP - N), (0, 31)))
    A0 = _agg_kernel_split(80)(g0a, g0b, ei)
    wl0p = padw(params["Wl0"], 128, 208)
    h1s, r1, s1, invd = _tc_layer(
        A0, None, None, None,
        xrs=(x, jnp.concatenate([padw(params["Wr0"], 128, 208),
                                 padw(params["W0"], 128, 208)], axis=1)),
        Wl=jnp.concatenate([wl0p[:80], wl0p[80:128],
                            jnp.zeros((32, 208), f32)], axis=0),
        bl=padv(params["bl0"], 208),
        b=padv(params["b0"], 208), ln=None,
        wcat=jnp.concatenate([padw(params["Wr1"], 208, 208),
                              padw(params["W1"], 208, 208)], axis=1),
        splits=[208, 208], out_h='split', first=True, cat_cols=True)

    A1 = _agg_kernel_split(112)(h1s[0], h1s[1], ei)
    wl1p = padw(params["Wl1"], 208, 208)
    g2, r2, s2 = _tc_layer(
        A1, r1, s1, invd,
        Wl=jnp.concatenate([wl1p[:112], wl1p[112:208],
                            jnp.zeros((16, 208), f32)], axis=0),
        bl=padv(params["bl1"], 208),
        b=padv(params["b1"], 208), ln=None,
        wcat=jnp.concatenate([padw(params["Wl2"], 208, 112),
                              padw(params["Wr2"], 208, 112),
                              padw(params["W2"], 208, 112)], axis=1),
        splits=[112, 112, 112], out_h=None, first=False, cat_cols=True,
        np_first_split=True)

    A2 = agg(g2)
    h3, r3, s3 = _tc_layer(
        A2, r2, s2, invd,
        Wl=None, bl=padv(params["bl2"], 112), b=padv(params["b2"], 112),
        ln=(padv(params["g3"], 112), padv(params["be3"], 112), 100),
        wcat=jnp.concatenate([padw(params["Wr3"], 112, 112),
                              padw(params["W3"], 112, 112)], axis=1),
        splits=[112, 112], out_h='plain', first=False)

    A3 = agg(h3)
    g4, r4, s4 = _tc_layer(
        A3, r3, s3, invd,
        Wl=padw(params["Wl3"], 112, 112), bl=padv(params["bl3"], 112),
        b=padv(params["b3"], 112), ln=None,
        wcat=jnp.concatenate([padw(params["Wl4"], 112, 64),
                              padw(params["Wr4"], 112, 64),
                              padw(params["W4"], 112, 64)], axis=1),
        splits=[64, 64, 64], out_h=None, first=False, np_first_split=True)

    A4 = agg(g4)
    h5, r5, s5 = _tc_layer(
        A4, r4, s4, invd,
        Wl=None, bl=padv(params["bl4"], 64), b=padv(params["b4"], 64),
        ln=None,
        wcat=jnp.concatenate([padw(params["Wr5"], 64, 64),
                              padw(params["W5"], 64, 64)], axis=1),
        splits=[64, 64], out_h='plain', first=False)

    A5 = agg(h5)
    g6, r6 = _tc_layer(
        A5, r5, s5, invd,
        Wl=padw(params["Wl5"], 64, 64), bl=padv(params["bl5"], 64),
        b=padv(params["b5"], 64),
        ln=(padv(params["g6"], 64), padv(params["be6"], 64), 50),
        wcat=jnp.concatenate([padw(params["Wl6"], 64, 32),
                              padw(params["Wr6"], 64, 32)], axis=1),
        splits=[32, 32], out_h=None, first=False, np_first_split=True)

    A6 = agg(g6)
    return _tc_final(A6, r6, invd, padv(params["bl6"], 32))
